# baseline (speedup 1.0000x reference)
import functools

import jax
import jax.numpy as jnp
from jax import lax
from jax.experimental import pallas as pl
from jax.experimental.pallas import tpu as pltpu
from jax.experimental.pallas import tpu_sc as plsc

_CODEBOOK = 1024
_DIM = 256
_BM = 1024
_NC, _NS = 2, 16
_NW = _NC * _NS
_NSLICE = 3


def _dist_block(x_ref, e_ref, ind_ref, loss_ref):
    i = pl.program_id(0)
    xb = x_ref[...]
    emb = e_ref[...]

    fsq = jnp.sum(xb * xb, axis=1)
    esq = jnp.sum(emb * emb, axis=1)
    cross_t = jax.lax.dot_general(
        emb, xb, (((1,), (1,)), ((), ())),
        preferred_element_type=jnp.float32,
        precision=jax.lax.Precision.DEFAULT,
    )
    fsq_t = jax.lax.transpose(fsq.reshape(_BM, 1), (1, 0))
    dist = (2.0 * cross_t - fsq_t) - esq[:, None]

    vals = dist
    half = _CODEBOOK // 2
    while half >= 8:
        vals = jnp.maximum(vals[:half], vals[half:])
        half //= 2
    m = jnp.max(vals, axis=0)

    iota0 = jax.lax.broadcasted_iota(jnp.int32, dist.shape, 0)
    sel = jnp.where(dist == m[None, :], iota0, _CODEBOOK)
    half = _CODEBOOK // 2
    while half >= 8:
        sel = jnp.minimum(sel[:half], sel[half:])
        half //= 2
    ind = jnp.min(sel, axis=0)
    ind_ref[0, 0, :] = ind

    part = -jnp.sum(m)

    @pl.when(i == 0)
    def _():
        loss_ref[0, 0] = 0.0

    loss_ref[0, 0] += part


def _nearest_codes(x2d, embed):
    tokens = x2d.shape[0]
    grid = tokens // _BM
    ind, loss_part = pl.pallas_call(
        _dist_block,
        grid=(grid,),
        in_specs=[
            pl.BlockSpec((_BM, _DIM), lambda i: (i, 0)),
            pl.BlockSpec((_CODEBOOK, _DIM), lambda i: (0, 0)),
        ],
        out_specs=[
            pl.BlockSpec((1, 1, _BM), lambda i: (i, 0, 0)),
            pl.BlockSpec(memory_space=pltpu.SMEM, block_shape=(1, 1),
                         index_map=lambda i: (0, 0)),
        ],
        out_shape=[
            jax.ShapeDtypeStruct((grid, 1, _BM), jnp.int32),
            jax.ShapeDtypeStruct((1, 1), jnp.float32),
        ],
    )(x2d, embed)
    return ind.reshape(tokens), loss_part.reshape(())


def _make_sc_gather(tokens):
    b_per_w = tokens // _NW
    mesh = plsc.VectorSubcoreMesh(core_axis_name="c", subcore_axis_name="s")

    @functools.partial(
        pl.kernel, mesh=mesh,
        out_type=jax.ShapeDtypeStruct((tokens, _DIM), jnp.float32),
        scratch_types=[
            pltpu.VMEM((b_per_w,), jnp.int32),
            pltpu.VMEM((b_per_w, _DIM), jnp.float32),
            pltpu.SemaphoreType.DMA,
        ],
    )
    def _gather(table_hbm, idx_hbm, out_hbm, idx_v, rows_v, sem):
        wid = lax.axis_index("s") * _NC + lax.axis_index("c")
        base = wid * b_per_w
        pltpu.sync_copy(idx_hbm.at[pl.ds(base, b_per_w)], idx_v)
        pltpu.async_copy(table_hbm.at[idx_v], rows_v, sem).wait()
        pltpu.sync_copy(rows_v, out_hbm.at[pl.ds(base, b_per_w)])

    return _gather


@jax.jit
def kernel(x, embed):
    b, n, d = x.shape
    tokens = b * n
    x2d = x.reshape(tokens, d)
    sl = tokens // _NSLICE

    sc_gather = _make_sc_gather(sl)
    inds, qs, parts = [], [], []
    for s in range(_NSLICE):
        ind_s, part_s = _nearest_codes(x2d[s * sl:(s + 1) * sl], embed)
        q_s = sc_gather(embed, ind_s)
        inds.append(ind_s)
        qs.append(q_s)
        parts.append(part_s)

    q = jnp.concatenate(qs, axis=0)
    ind = jnp.concatenate(inds, axis=0)
    loss = (parts[0] + parts[1] + parts[2]) * jnp.float32(1.0 / (tokens * d))
    return (q.reshape(b, n, d), ind.reshape(b, n), loss)


# speedup vs baseline: 1.3697x; 1.3697x over previous
import functools

import jax
import jax.numpy as jnp
from jax import lax
from jax.experimental import pallas as pl
from jax.experimental.pallas import tpu as pltpu
from jax.experimental.pallas import tpu_sc as plsc

_CODEBOOK = 1024
_DIM = 256
_BM = 1024
_NC, _NS = 2, 16
_NW = _NC * _NS
_CHUNK = 96


def _dist_block(x_ref, e_ref, ind_ref, loss_ref):
    i = pl.program_id(0)
    g = pl.num_programs(0)
    xb = x_ref[...]
    emb = e_ref[...]

    fsq = jnp.sum(xb * xb, axis=1)
    esq = jnp.sum(emb * emb, axis=1)
    cross_t = jax.lax.dot_general(
        emb, xb, (((1,), (1,)), ((), ())),
        preferred_element_type=jnp.float32,
        precision=jax.lax.Precision.DEFAULT,
    )
    fsq_t = jax.lax.transpose(fsq.reshape(_BM, 1), (1, 0))
    dist = (2.0 * cross_t - fsq_t) - esq[:, None]

    vals = dist
    half = _CODEBOOK // 2
    while half >= 8:
        vals = jnp.maximum(vals[:half], vals[half:])
        half //= 2
    m = jnp.max(vals, axis=0)

    iota0 = jax.lax.broadcasted_iota(jnp.int32, dist.shape, 0)
    sel = jnp.where(dist == m[None, :], iota0, _CODEBOOK)
    half = _CODEBOOK // 2
    while half >= 8:
        sel = jnp.minimum(sel[:half], sel[half:])
        half //= 2
    ind = jnp.min(sel, axis=0)
    ind_ref[0, 0, :] = ind

    part = -jnp.sum(m)

    @pl.when(i == 0)
    def _():
        loss_ref[0, 0] = 0.0

    loss_ref[0, 0] += part

    @pl.when(i == g - 1)
    def _():
        loss_ref[0, 0] = loss_ref[0, 0] / jnp.float32(_BM * g * _DIM)


def _nearest_codes(x2d, embed):
    tokens = x2d.shape[0]
    grid = tokens // _BM
    ind, loss = pl.pallas_call(
        _dist_block,
        grid=(grid,),
        in_specs=[
            pl.BlockSpec((_BM, _DIM), lambda i: (i, 0)),
            pl.BlockSpec((_CODEBOOK, _DIM), lambda i: (0, 0)),
        ],
        out_specs=[
            pl.BlockSpec((1, 1, _BM), lambda i: (i, 0, 0)),
            pl.BlockSpec(memory_space=pltpu.SMEM, block_shape=(1, 1),
                         index_map=lambda i: (0, 0)),
        ],
        out_shape=[
            jax.ShapeDtypeStruct((grid, 1, _BM), jnp.int32),
            jax.ShapeDtypeStruct((1, 1), jnp.float32),
        ],
    )(x2d, embed)
    return ind.reshape(tokens), loss.reshape(())


def _make_sc_gather(tokens):
    b_per_w = tokens // _NW
    n_chunks = b_per_w // _CHUNK
    mesh = plsc.VectorSubcoreMesh(core_axis_name="c", subcore_axis_name="s")

    @functools.partial(
        pl.kernel, mesh=mesh,
        out_type=jax.ShapeDtypeStruct((tokens, _DIM), jnp.float32),
        scratch_types=[
            pltpu.VMEM((b_per_w,), jnp.int32),
            pltpu.VMEM((b_per_w, _DIM), jnp.float32),
            pltpu.SemaphoreType.DMA,
        ],
    )
    def _gather(table_hbm, idx_hbm, out_hbm, idx_v, rows_v, sem):
        wid = lax.axis_index("s") * _NC + lax.axis_index("c")
        base = wid * b_per_w
        pltpu.sync_copy(idx_hbm.at[pl.ds(base, b_per_w)], idx_v)
        copies = [
            pltpu.async_copy(
                table_hbm.at[idx_v.at[pl.ds(c * _CHUNK, _CHUNK)]],
                rows_v.at[pl.ds(c * _CHUNK, _CHUNK), :],
                sem,
            )
            for c in range(n_chunks)
        ]
        for cp in copies:
            cp.wait()
        pltpu.sync_copy(rows_v, out_hbm.at[pl.ds(base, b_per_w)])

    return _gather


@jax.jit
def kernel(x, embed):
    b, n, d = x.shape
    tokens = b * n
    x2d = x.reshape(tokens, d)

    ind, loss = _nearest_codes(x2d, embed)
    q = _make_sc_gather(tokens)(embed, ind)

    return (q.reshape(b, n, d), ind.reshape(b, n), loss)
